# baseline (speedup 1.0000x reference)
import functools

import jax
import jax.numpy as jnp
from jax import lax
from jax.experimental import pallas as pl
from jax.experimental.pallas import tpu as pltpu
from jax.experimental.pallas import tpu_sc as plsc

HIDDEN_DIM = 64
PADDED_DIM = 128
SEQ_PAD = 56
NUM_CORES = 2
NUM_SUBCORES = 16
NUM_WORKERS = NUM_CORES * NUM_SUBCORES
CHUNK = 128
NBUF = 4
IGNORED_ID = -1


def kernel(input_ids, embed_tokens_weight):
    batch, seq = input_ids.shape
    vocab = embed_tokens_weight.shape[0]
    num_idx = batch * SEQ_PAD
    per_worker = num_idx // NUM_WORKERS
    n_iter = per_worker // CHUNK

    dummy = jnp.full((batch, SEQ_PAD - seq), IGNORED_ID, input_ids.dtype)
    flat_ids = jnp.concatenate([input_ids, dummy], axis=1).reshape(num_idx)

    half = vocab // 2
    pad_cfg = ((0, 0), (0, PADDED_DIM - HIDDEN_DIM))
    p0 = jnp.pad(embed_tokens_weight[:half], pad_cfg)
    p1 = jnp.pad(embed_tokens_weight[half:], pad_cfg)
    buf = jnp.zeros((vocab, PADDED_DIM), jnp.float32)
    table128 = lax.dynamic_update_slice(
        lax.dynamic_update_slice(buf, p0, (0, 0)), p1, (half, 0)
    )

    mesh = plsc.VectorSubcoreMesh(core_axis_name="c", subcore_axis_name="s")

    @functools.partial(
        pl.kernel,
        mesh=mesh,
        out_type=jax.ShapeDtypeStruct((num_idx, PADDED_DIM), jnp.float32),
        scratch_types=[
            pltpu.VMEM((NBUF, CHUNK), jnp.int32),
            pltpu.VMEM((NBUF, CHUNK, PADDED_DIM), jnp.float32),
            pltpu.SemaphoreType.DMA((NBUF,)),
            pltpu.SemaphoreType.DMA((NBUF,)),
            pltpu.SemaphoreType.DMA((NBUF,)),
        ],
    )
    def gather_kernel(table_hbm, idx_hbm, out_hbm, idx_v, rows_v, sem_idx,
                      sem_gat, sem_out):
        wid = lax.axis_index("s") * NUM_CORES + lax.axis_index("c")
        base = wid * per_worker

        def idx_copy(it, buf):
            return pltpu.make_async_copy(
                idx_hbm.at[pl.ds(base + it * CHUNK, CHUNK)],
                idx_v.at[buf],
                sem_idx.at[buf],
            )

        def gat_copy(buf):
            return pltpu.make_async_copy(
                table_hbm.at[
                    plsc.Indices(idx_v.at[buf], ignored_value=IGNORED_ID)
                ],
                rows_v.at[buf],
                sem_gat.at[buf],
            )

        def out_copy(it, buf):
            return pltpu.make_async_copy(
                rows_v.at[buf],
                out_hbm.at[pl.ds(base + it * CHUNK, CHUNK)],
                sem_out.at[buf],
            )

        idx_copy(0, 0).start()

        @pl.loop(0, n_iter, step=NBUF)
        def _(i):
            for b in range(NBUF):
                it = i + b
                idx_copy(it, b).wait()
                @pl.when(it >= NBUF)
                def _():
                    out_copy(it - NBUF, b).wait()

                gat_copy(b).start()

                @pl.when(it + 1 < n_iter)
                def _():
                    idx_copy(it + 1, (b + 1) % NBUF).start()

                @pl.when(it >= 1)
                def _():
                    gat_copy((b - 1) % NBUF).wait()
                    out_copy(it - 1, (b - 1) % NBUF).start()

        last = n_iter - 1
        gat_copy(last % NBUF).wait()
        out_copy(last, last % NBUF).start()
        for j in range(NBUF - 1, 0, -1):
            out_copy(last - j, (last - j) % NBUF).wait()
        out_copy(last, last % NBUF).wait()

    fat = gather_kernel(table128, flat_ids)
    fat3 = fat.reshape(batch, SEQ_PAD, PADDED_DIM)
    return fat3[:, :seq, :HIDDEN_DIM]


# speedup vs baseline: 1.3617x; 1.3617x over previous
import functools

import jax
import jax.numpy as jnp
from jax import lax
from jax.experimental import pallas as pl
from jax.experimental.pallas import tpu as pltpu
from jax.experimental.pallas import tpu_sc as plsc

HIDDEN_DIM = 64
PADDED_DIM = 128
SEQ_PAD = 56
NUM_CORES = 2
NUM_SUBCORES = 16
NUM_WORKERS = NUM_CORES * NUM_SUBCORES
CHUNK = 128
NBUF = 4
IGNORED_ID = -1


def kernel(input_ids, embed_tokens_weight):
    batch, seq = input_ids.shape
    vocab = embed_tokens_weight.shape[0]
    num_idx = batch * SEQ_PAD
    per_worker = num_idx // NUM_WORKERS
    n_iter = per_worker // CHUNK

    dummy = jnp.full((batch, SEQ_PAD - seq), IGNORED_ID, input_ids.dtype)
    flat_ids = jnp.concatenate([input_ids, dummy], axis=1).reshape(num_idx)

    table128 = jnp.pad(
        embed_tokens_weight, ((0, 0), (0, PADDED_DIM - HIDDEN_DIM))
    )

    mesh = plsc.VectorSubcoreMesh(core_axis_name="c", subcore_axis_name="s")

    @functools.partial(
        pl.kernel,
        mesh=mesh,
        out_type=jax.ShapeDtypeStruct((num_idx, PADDED_DIM), jnp.float32),
        scratch_types=[
            pltpu.VMEM((NBUF, CHUNK), jnp.int32),
            pltpu.VMEM((NBUF, CHUNK, PADDED_DIM), jnp.float32),
            pltpu.SemaphoreType.DMA((NBUF,)),
            pltpu.SemaphoreType.DMA((NBUF,)),
            pltpu.SemaphoreType.DMA((NBUF,)),
        ],
    )
    def gather_kernel(table_hbm, idx_hbm, out_hbm, idx_v, rows_v, sem_idx,
                      sem_gat, sem_out):
        wid = lax.axis_index("s") * NUM_CORES + lax.axis_index("c")
        base = wid * per_worker

        def idx_copy(it, buf):
            return pltpu.make_async_copy(
                idx_hbm.at[pl.ds(base + it * CHUNK, CHUNK)],
                idx_v.at[buf],
                sem_idx.at[buf],
            )

        def gat_copy(buf):
            return pltpu.make_async_copy(
                table_hbm.at[
                    plsc.Indices(idx_v.at[buf], ignored_value=IGNORED_ID)
                ],
                rows_v.at[buf],
                sem_gat.at[buf],
            )

        def out_copy(it, buf):
            return pltpu.make_async_copy(
                rows_v.at[buf],
                out_hbm.at[pl.ds(base + it * CHUNK, CHUNK)],
                sem_out.at[buf],
            )

        idx_copy(0, 0).start()

        @pl.loop(0, n_iter, step=NBUF)
        def _(i):
            for b in range(NBUF):
                it = i + b
                idx_copy(it, b).wait()
                @pl.when(it >= NBUF)
                def _():
                    out_copy(it - NBUF, b).wait()

                gat_copy(b).start()

                @pl.when(it + 1 < n_iter)
                def _():
                    idx_copy(it + 1, (b + 1) % NBUF).start()

                @pl.when(it >= 2)
                def _():
                    gat_copy((b - 2) % NBUF).wait()
                    out_copy(it - 2, (b - 2) % NBUF).start()

        last = n_iter - 1
        gat_copy((last - 1) % NBUF).wait()
        out_copy(last - 1, (last - 1) % NBUF).start()
        gat_copy(last % NBUF).wait()
        out_copy(last, last % NBUF).start()
        for j in range(NBUF - 1, -1, -1):
            out_copy(last - j, (last - j) % NBUF).wait()

    fat = gather_kernel(table128, flat_ids)
    fat3 = fat.reshape(batch, SEQ_PAD, PADDED_DIM)
    return fat3[:, :seq, :HIDDEN_DIM]
